# Initial kernel scaffold; baseline (speedup 1.0000x reference)
import jax
import jax.numpy as jnp
from jax.experimental import pallas as pl


def kernel(embedding, src_id, dst_id, edge_feats, W, b):
    raise NotImplementedError("write your pallas kernel here")


# speedup vs baseline: 1.0011x; 1.0011x over previous
import functools

import jax
import jax.numpy as jnp
from jax import lax
from jax.experimental import pallas as pl
from jax.experimental.pallas import tpu as pltpu
from jax.experimental.pallas import tpu_sc as plsc

N_NODES = 10000
N_EDGES = 320000
D_EMB = 128
D_FEAT = 6

NUM_CORES = 2
NUM_SUBCORES = 16
NUM_WORKERS = NUM_CORES * NUM_SUBCORES
EDGES_PER_WORKER = N_EDGES // NUM_WORKERS
CHUNK = 80
NUM_CHUNKS = EDGES_PER_WORKER // CHUNK
GROUPS = CHUNK // 16

_FD_BLOCK = 40000


def _scale_body(e_ref, w_ref, o_ref):
    o_ref[...] = e_ref[...] * w_ref[...]


def _scale_table(embedding, w128):
    return pl.pallas_call(
        _scale_body,
        out_shape=jax.ShapeDtypeStruct((N_NODES, D_EMB), jnp.float32),
    )(embedding, w128)


def _featdot_body(f_ref, sw_ref, o_ref):
    o_ref[...] = jax.lax.dot(f_ref[...], sw_ref[...],
                             precision=jax.lax.Precision.HIGHEST)


def _featdot(feats_v2, sw):
    n_rows = N_EDGES // 16
    blk = n_rows // 10
    return pl.pallas_call(
        _featdot_body,
        grid=(10,),
        in_specs=[
            pl.BlockSpec((blk, 128), lambda i: (i, 0)),
            pl.BlockSpec((128, 16), lambda i: (0, 0)),
        ],
        out_specs=pl.BlockSpec((blk, 16), lambda i: (i, 0)),
        out_shape=jax.ShapeDtypeStruct((n_rows, 16), jnp.float32),
    )(feats_v2, sw)


_mesh = plsc.VectorSubcoreMesh(core_axis_name="c", subcore_axis_name="s")


@functools.partial(
    pl.kernel,
    mesh=_mesh,
    out_type=jax.ShapeDtypeStruct((N_EDGES,), jnp.float32),
    compiler_params=pltpu.CompilerParams(needs_layout_passes=False),
    scratch_types=[
        pltpu.VMEM((CHUNK,), jnp.int32),
        pltpu.VMEM((CHUNK,), jnp.int32),
        pltpu.VMEM((CHUNK,), jnp.float32),
        pltpu.VMEM((CHUNK, D_EMB), jnp.float32),
        pltpu.VMEM((CHUNK, D_EMB), jnp.float32),
        pltpu.VMEM((CHUNK,), jnp.float32),
        pltpu.SemaphoreType.DMA,
    ],
)
def _edge_kernel(es_hbm, e_hbm, src_hbm, dst_hbm, fc_hbm, out_hbm,
                 sidx_v, didx_v, fc_v, srow_v, drow_v, o_v, sem):
    wid = lax.axis_index("s") * NUM_CORES + lax.axis_index("c")
    base = wid * EDGES_PER_WORKER
    lanes = lax.iota(jnp.int32, 16)

    def chunk_body(ci, carry):
        cb = base + ci * CHUNK
        pltpu.sync_copy(src_hbm.at[pl.ds(cb, CHUNK)], sidx_v)
        pltpu.sync_copy(dst_hbm.at[pl.ds(cb, CHUNK)], didx_v)
        pltpu.sync_copy(fc_hbm.at[pl.ds(cb, CHUNK)], fc_v)
        cp_s = pltpu.async_copy(es_hbm.at[sidx_v], srow_v, sem)
        cp_d = pltpu.async_copy(e_hbm.at[didx_v], drow_v, sem)
        cp_s.wait()
        cp_d.wait()

        def group_body(g, gcarry):
            rvec = g * 16 + lanes
            acc = fc_v[pl.ds(g * 16, 16)]
            for k in range(D_EMB):
                colv = jnp.full((16,), k, jnp.int32)
                sv = plsc.load_gather(srow_v, [rvec, colv])
                dv = plsc.load_gather(drow_v, [rvec, colv])
                acc = acc + sv * dv
            o_v[pl.ds(g * 16, 16)] = 1.0 / (1.0 + jnp.exp(-acc))
            return gcarry

        lax.fori_loop(0, GROUPS, group_body, 0)
        pltpu.sync_copy(o_v, out_hbm.at[pl.ds(cb, CHUNK)])
        return carry

    lax.fori_loop(0, NUM_CHUNKS, chunk_body, 0)


def kernel(embedding, src_id, dst_id, edge_feats, W, b):
    w128 = W[:D_EMB, 0].reshape(1, D_EMB)
    es = _scale_table(embedding, w128)
    feats_p = jnp.concatenate(
        [edge_feats,
         jnp.ones((N_EDGES, 1), jnp.float32),
         jnp.zeros((N_EDGES, 1), jnp.float32)], axis=1)
    w8 = jnp.concatenate([W[D_EMB:, 0], b, jnp.zeros((1,), jnp.float32)])
    sw = jnp.kron(jnp.eye(16, dtype=jnp.float32), w8.reshape(8, 1))
    fc = _featdot(feats_p.reshape(N_EDGES // 16, 128), sw).reshape(-1)
    out = _edge_kernel(
        es, embedding,
        src_id.astype(jnp.int32), dst_id.astype(jnp.int32), fc)
    return out.reshape(N_EDGES, 1)


# speedup vs baseline: 1.2031x; 1.2018x over previous
import functools

import jax
import jax.numpy as jnp
from jax import lax
from jax.experimental import pallas as pl
from jax.experimental.pallas import tpu as pltpu
from jax.experimental.pallas import tpu_sc as plsc

N_NODES = 10000
N_EDGES = 320000
D_EMB = 128
D_FEAT = 6

NUM_CORES = 2
NUM_SUBCORES = 16
NUM_WORKERS = NUM_CORES * NUM_SUBCORES
EDGES_PER_WORKER = N_EDGES // NUM_WORKERS
CHUNK = 80
NUM_CHUNKS = EDGES_PER_WORKER // CHUNK
GROUPS = CHUNK // 16
NBUF = 5
ROWS = 2 * CHUNK


def _scale_body(e_ref, w_ref, o_ref):
    o_ref[...] = e_ref[...] * w_ref[...]


def _scale_table(embedding, w128):
    return pl.pallas_call(
        _scale_body,
        out_shape=jax.ShapeDtypeStruct((N_NODES, D_EMB), jnp.float32),
    )(embedding, w128)


def _featdot_body(f_ref, sw_ref, o_ref):
    o_ref[...] = jax.lax.dot(f_ref[...], sw_ref[...],
                             precision=jax.lax.Precision.HIGHEST)


def _featdot(feats_v2, sw):
    n_rows = N_EDGES // 16
    blk = n_rows // 10
    return pl.pallas_call(
        _featdot_body,
        grid=(10,),
        in_specs=[
            pl.BlockSpec((blk, 128), lambda i: (i, 0)),
            pl.BlockSpec((128, 16), lambda i: (0, 0)),
        ],
        out_specs=pl.BlockSpec((blk, 16), lambda i: (i, 0)),
        out_shape=jax.ShapeDtypeStruct((n_rows, 16), jnp.float32),
    )(feats_v2, sw)


_mesh = plsc.VectorSubcoreMesh(core_axis_name="c", subcore_axis_name="s")


@functools.partial(
    pl.kernel,
    mesh=_mesh,
    out_type=jax.ShapeDtypeStruct((N_EDGES,), jnp.float32),
    compiler_params=pltpu.CompilerParams(needs_layout_passes=False),
    scratch_types=[
        pltpu.VMEM((NUM_CHUNKS * ROWS,), jnp.int32),
        pltpu.VMEM((NBUF, CHUNK), jnp.float32),
        pltpu.VMEM((NBUF, ROWS, D_EMB), jnp.float32),
        pltpu.VMEM((NBUF, CHUNK), jnp.float32),
        pltpu.SemaphoreType.DMA((NBUF,)),
        pltpu.SemaphoreType.DMA((NBUF,)),
    ],
)
def _edge_kernel(tbl_hbm, ids_hbm, fc_hbm, out_hbm,
                 ids_v, fc_v, rows_v, ob_v, sem_g, sem_o):
    wid = lax.axis_index("s") * NUM_CORES + lax.axis_index("c")
    ebase = wid * EDGES_PER_WORKER
    pltpu.sync_copy(ids_hbm.at[pl.ds(wid * NUM_CHUNKS * ROWS,
                                     NUM_CHUNKS * ROWS)], ids_v)
    lanes = lax.iota(jnp.int32, 16)

    def fc_copy(i, s):
        return pltpu.make_async_copy(
            fc_hbm.at[pl.ds(ebase + i * CHUNK, CHUNK)], fc_v.at[s],
            sem_g.at[s])

    def row_gather(i, s):
        return pltpu.make_async_copy(
            tbl_hbm.at[ids_v.at[pl.ds(i * ROWS, ROWS)]], rows_v.at[s],
            sem_g.at[s])

    def out_copy(i, s):
        return pltpu.make_async_copy(
            ob_v.at[s], out_hbm.at[pl.ds(ebase + i * CHUNK, CHUNK)],
            sem_o.at[s])

    def issue(i, s):
        fc_copy(i, s).start()
        row_gather(i, s).start()

    def compute(i, s):
        rows2d = rows_v.at[s]

        def group_body(g, gcarry):
            rvec = g * 16 + lanes
            svec = rvec
            dvec = rvec + CHUNK

            def k_body(k4, accs):
                a0, a1, a2, a3 = accs
                acc4 = [a0, a1, a2, a3]
                for u in range(32):
                    k = k4 * 32 + u
                    colv = jnp.full((16,), k, jnp.int32)
                    sv = plsc.load_gather(rows2d, [svec, colv])
                    dv = plsc.load_gather(rows2d, [dvec, colv])
                    acc4[u % 4] = acc4[u % 4] + sv * dv
                return tuple(acc4)

            z = jnp.zeros((16,), jnp.float32)
            a0, a1, a2, a3 = lax.fori_loop(0, 4, k_body, (z, z, z, z))
            acc = ((a0 + a1) + (a2 + a3)) + fc_v.at[s][pl.ds(g * 16, 16)]
            ob_v.at[s][pl.ds(g * 16, 16)] = 1.0 / (1.0 + jnp.exp(-acc))
            return gcarry

        lax.fori_loop(0, GROUPS, group_body, 0)

    issue(0, 0)
    issue(1, 1)

    def j_body(j, carry):
        for s in range(NBUF):
            i = j * NBUF + s

            @pl.when(j >= 1)
            def _():
                out_copy(i - NBUF, s).wait()

            fc_copy(i, s).wait()
            row_gather(i, s).wait()
            s2 = (s + 2) % NBUF
            if s < NBUF - 2:
                issue(i + 2, s2)
            else:
                @pl.when(j <= NUM_CHUNKS // NBUF - 2)
                def _():
                    issue(i + 2, s2)
            compute(i, s)
            out_copy(i, s).start()
        return carry

    lax.fori_loop(0, NUM_CHUNKS // NBUF, j_body, 0)
    for s in range(NBUF):
        out_copy(NUM_CHUNKS - NBUF + s, s).wait()


def kernel(embedding, src_id, dst_id, edge_feats, W, b):
    w128 = W[:D_EMB, 0].reshape(1, D_EMB)
    es = _scale_table(embedding, w128)
    tbl = jnp.concatenate([es, embedding], axis=0)
    ids = jnp.stack(
        [src_id.astype(jnp.int32).reshape(-1, CHUNK),
         dst_id.astype(jnp.int32).reshape(-1, CHUNK) + N_NODES],
        axis=1).reshape(-1)
    feats_p = jnp.concatenate(
        [edge_feats,
         jnp.ones((N_EDGES, 1), jnp.float32),
         jnp.zeros((N_EDGES, 1), jnp.float32)], axis=1)
    w8 = jnp.concatenate([W[D_EMB:, 0], b, jnp.zeros((1,), jnp.float32)])
    sw = jnp.kron(jnp.eye(16, dtype=jnp.float32), w8.reshape(8, 1))
    fc = _featdot(feats_p.reshape(N_EDGES // 16, 128), sw).reshape(-1)
    out = _edge_kernel(tbl, ids, fc)
    return out.reshape(N_EDGES, 1)


# speedup vs baseline: 3.6842x; 3.0621x over previous
import functools

import jax
import jax.numpy as jnp
from jax import lax
from jax.experimental import pallas as pl
from jax.experimental.pallas import tpu as pltpu
from jax.experimental.pallas import tpu_sc as plsc

N_NODES = 10000
N_EDGES = 320000
D_EMB = 128
D_FEAT = 6

NUM_CORES = 2
NUM_SUBCORES = 16
NUM_WORKERS = NUM_CORES * NUM_SUBCORES
EDGES_PER_WORKER = N_EDGES // NUM_WORKERS
CHUNK = 80
NUM_CHUNKS = EDGES_PER_WORKER // CHUNK
GROUPS = CHUNK // 16
NBUF = 5
ROWS = 2 * CHUNK


def _scale_body(e_ref, w_ref, o_ref):
    o_ref[...] = e_ref[...] * w_ref[...]


def _scale_table(embedding, w128):
    return pl.pallas_call(
        _scale_body,
        out_shape=jax.ShapeDtypeStruct((N_NODES, D_EMB), jnp.float32),
    )(embedding, w128)


def _featdot_body(f_ref, sw_ref, o_ref):
    o_ref[...] = jax.lax.dot(f_ref[...], sw_ref[...],
                             precision=jax.lax.Precision.HIGHEST)


def _featdot(feats_v2, sw):
    n_rows = N_EDGES // 16
    blk = n_rows // 10
    return pl.pallas_call(
        _featdot_body,
        grid=(10,),
        in_specs=[
            pl.BlockSpec((blk, 128), lambda i: (i, 0)),
            pl.BlockSpec((128, 16), lambda i: (0, 0)),
        ],
        out_specs=pl.BlockSpec((blk, 16), lambda i: (i, 0)),
        out_shape=jax.ShapeDtypeStruct((n_rows, 16), jnp.float32),
    )(feats_v2, sw)


_mesh = plsc.VectorSubcoreMesh(core_axis_name="c", subcore_axis_name="s")


@functools.partial(
    pl.kernel,
    mesh=_mesh,
    out_type=jax.ShapeDtypeStruct((N_EDGES,), jnp.float32),
    compiler_params=pltpu.CompilerParams(needs_layout_passes=False),
    scratch_types=[
        pltpu.VMEM((NUM_CHUNKS * ROWS,), jnp.int32),
        pltpu.VMEM((NBUF, CHUNK), jnp.float32),
        pltpu.VMEM((NBUF, ROWS, D_EMB), jnp.float32),
        pltpu.VMEM((NBUF, CHUNK), jnp.float32),
        pltpu.VMEM((16,), jnp.float32),
        pltpu.SemaphoreType.DMA((NBUF,)),
        pltpu.SemaphoreType.DMA((NBUF,)),
    ],
)
def _edge_kernel(tbl_hbm, ids_hbm, fc_hbm, out_hbm,
                 ids_v, fc_v, rows_v, ob_v, tmp_v, sem_g, sem_o):
    wid = lax.axis_index("s") * NUM_CORES + lax.axis_index("c")
    ebase = wid * EDGES_PER_WORKER
    pltpu.sync_copy(ids_hbm.at[pl.ds(wid * NUM_CHUNKS * ROWS,
                                     NUM_CHUNKS * ROWS)], ids_v)
    lanes = lax.iota(jnp.int32, 16)

    def fc_copy(i, s):
        return pltpu.make_async_copy(
            fc_hbm.at[pl.ds(ebase + i * CHUNK, CHUNK)], fc_v.at[s],
            sem_g.at[s])

    def row_gather(i, s):
        return pltpu.make_async_copy(
            tbl_hbm.at[ids_v.at[pl.ds(i * ROWS, ROWS)]], rows_v.at[s],
            sem_g.at[s])

    def out_copy(i, s):
        return pltpu.make_async_copy(
            ob_v.at[s], out_hbm.at[pl.ds(ebase + i * CHUNK, CHUNK)],
            sem_o.at[s])

    def issue(i, s):
        fc_copy(i, s).start()
        row_gather(i, s).start()

    def compute(i, s):
        rows2d = rows_v.at[s]
        last_lane = lanes == 15

        def group_body(g, gcarry):
            gb = g * 16
            for e in range(16):
                srow = rows2d.at[gb + e]
                drow = rows2d.at[gb + CHUNK + e]
                prods = [srow[pl.ds(u * 16, 16)] * drow[pl.ds(u * 16, 16)]
                         for u in range(8)]
                p01, p23 = prods[0] + prods[1], prods[2] + prods[3]
                p45, p67 = prods[4] + prods[5], prods[6] + prods[7]
                partial = (p01 + p23) + (p45 + p67)
                csum = plsc.cumsum(partial)
                plsc.store_scatter(tmp_v, [jnp.full((16,), e, jnp.int32)],
                                   csum, mask=last_lane)
            acc = tmp_v[...] + fc_v.at[s][pl.ds(gb, 16)]
            ob_v.at[s][pl.ds(gb, 16)] = 1.0 / (1.0 + jnp.exp(-acc))
            return gcarry

        lax.fori_loop(0, GROUPS, group_body, 0)

    issue(0, 0)
    issue(1, 1)

    def j_body(j, carry):
        for s in range(NBUF):
            i = j * NBUF + s

            @pl.when(j >= 1)
            def _():
                out_copy(i - NBUF, s).wait()

            fc_copy(i, s).wait()
            row_gather(i, s).wait()
            s2 = (s + 2) % NBUF
            if s < NBUF - 2:
                issue(i + 2, s2)
            else:
                @pl.when(j <= NUM_CHUNKS // NBUF - 2)
                def _():
                    issue(i + 2, s2)
            compute(i, s)
            out_copy(i, s).start()
        return carry

    lax.fori_loop(0, NUM_CHUNKS // NBUF, j_body, 0)
    for s in range(NBUF):
        out_copy(NUM_CHUNKS - NBUF + s, s).wait()


def kernel(embedding, src_id, dst_id, edge_feats, W, b):
    w128 = W[:D_EMB, 0].reshape(1, D_EMB)
    es = _scale_table(embedding, w128)
    tbl = jnp.concatenate([es, embedding], axis=0)
    ids = jnp.stack(
        [src_id.astype(jnp.int32).reshape(-1, CHUNK),
         dst_id.astype(jnp.int32).reshape(-1, CHUNK) + N_NODES],
        axis=1).reshape(-1)
    feats_p = jnp.concatenate(
        [edge_feats,
         jnp.ones((N_EDGES, 1), jnp.float32),
         jnp.zeros((N_EDGES, 1), jnp.float32)], axis=1)
    w8 = jnp.concatenate([W[D_EMB:, 0], b, jnp.zeros((1,), jnp.float32)])
    sw = jnp.kron(jnp.eye(16, dtype=jnp.float32), w8.reshape(8, 1))
    fc = _featdot(feats_p.reshape(N_EDGES // 16, 128), sw).reshape(-1)
    out = _edge_kernel(tbl, ids, fc)
    return out.reshape(N_EDGES, 1)


# speedup vs baseline: 4.0792x; 1.1072x over previous
import functools

import jax
import jax.numpy as jnp
from jax import lax
from jax.experimental import pallas as pl
from jax.experimental.pallas import tpu as pltpu
from jax.experimental.pallas import tpu_sc as plsc

N_NODES = 10000
N_EDGES = 320000
D_EMB = 128
D_FEAT = 6

NUM_CORES = 2
NUM_SUBCORES = 16
NUM_WORKERS = NUM_CORES * NUM_SUBCORES
EDGES_PER_WORKER = N_EDGES // NUM_WORKERS
CHUNK = 80
NUM_CHUNKS = EDGES_PER_WORKER // CHUNK
GROUPS = CHUNK // 16
NBUF = 5


def _scale_body(e_ref, w_ref, o_ref):
    o_ref[...] = e_ref[...] * w_ref[...]


def _scale_table(embedding, w128):
    return pl.pallas_call(
        _scale_body,
        out_shape=jax.ShapeDtypeStruct((N_NODES, D_EMB), jnp.float32),
    )(embedding, w128)


def _featdot_body(f_ref, sw_ref, b_ref, o_ref):
    o_ref[...] = jax.lax.dot(f_ref[...], sw_ref[...],
                             precision=jax.lax.Precision.HIGHEST) + b_ref[...]


def _featdot(feats_v2, sw, b16):
    n_rows = N_EDGES // 16
    blk = n_rows // 10
    return pl.pallas_call(
        _featdot_body,
        grid=(10,),
        in_specs=[
            pl.BlockSpec((blk, 96), lambda i: (i, 0)),
            pl.BlockSpec((96, 16), lambda i: (0, 0)),
            pl.BlockSpec((1, 16), lambda i: (0, 0)),
        ],
        out_specs=pl.BlockSpec((blk, 16), lambda i: (i, 0)),
        out_shape=jax.ShapeDtypeStruct((n_rows, 16), jnp.float32),
    )(feats_v2, sw, b16)


_mesh = plsc.VectorSubcoreMesh(core_axis_name="c", subcore_axis_name="s")


@functools.partial(
    pl.kernel,
    mesh=_mesh,
    out_type=jax.ShapeDtypeStruct((N_EDGES,), jnp.float32),
    compiler_params=pltpu.CompilerParams(needs_layout_passes=False),
    scratch_types=[
        pltpu.VMEM((EDGES_PER_WORKER,), jnp.int32),
        pltpu.VMEM((EDGES_PER_WORKER,), jnp.int32),
        pltpu.VMEM((NBUF, CHUNK), jnp.float32),
        pltpu.VMEM((NBUF, 2 * CHUNK, D_EMB), jnp.float32),
        pltpu.VMEM((NBUF, CHUNK), jnp.float32),
        pltpu.VMEM((16,), jnp.float32),
        pltpu.SemaphoreType.DMA((NBUF,)),
        pltpu.SemaphoreType.DMA((NBUF,)),
    ],
)
def _edge_kernel(es_hbm, e_hbm, src_hbm, dst_hbm, fc_hbm, out_hbm,
                 sidx_v, didx_v, fc_v, rows_v, ob_v, tmp_v, sem_g, sem_o):
    wid = lax.axis_index("s") * NUM_CORES + lax.axis_index("c")
    ebase = wid * EDGES_PER_WORKER
    pltpu.sync_copy(src_hbm.at[pl.ds(ebase, EDGES_PER_WORKER)], sidx_v)
    pltpu.sync_copy(dst_hbm.at[pl.ds(ebase, EDGES_PER_WORKER)], didx_v)
    lanes = lax.iota(jnp.int32, 16)

    def fc_copy(i, s):
        return pltpu.make_async_copy(
            fc_hbm.at[pl.ds(ebase + i * CHUNK, CHUNK)], fc_v.at[s],
            sem_g.at[s])

    def src_gather(i, s):
        return pltpu.make_async_copy(
            es_hbm.at[sidx_v.at[pl.ds(i * CHUNK, CHUNK)]],
            rows_v.at[s].at[pl.ds(0, CHUNK)], sem_g.at[s])

    def dst_gather(i, s):
        return pltpu.make_async_copy(
            e_hbm.at[didx_v.at[pl.ds(i * CHUNK, CHUNK)]],
            rows_v.at[s].at[pl.ds(CHUNK, CHUNK)], sem_g.at[s])

    def out_copy(i, s):
        return pltpu.make_async_copy(
            ob_v.at[s], out_hbm.at[pl.ds(ebase + i * CHUNK, CHUNK)],
            sem_o.at[s])

    def issue(i, s):
        fc_copy(i, s).start()
        src_gather(i, s).start()
        dst_gather(i, s).start()

    def wait_in(i, s):
        fc_copy(i, s).wait()
        src_gather(i, s).wait()
        dst_gather(i, s).wait()

    def compute(i, s):
        rows2d = rows_v.at[s]
        last_lane = lanes == 15

        def group_body(g, gcarry):
            gb = g * 16
            for e in range(16):
                srow = rows2d.at[gb + e]
                drow = rows2d.at[gb + CHUNK + e]
                prods = [srow[pl.ds(u * 16, 16)] * drow[pl.ds(u * 16, 16)]
                         for u in range(8)]
                p01, p23 = prods[0] + prods[1], prods[2] + prods[3]
                p45, p67 = prods[4] + prods[5], prods[6] + prods[7]
                partial = (p01 + p23) + (p45 + p67)
                csum = plsc.cumsum(partial)
                plsc.store_scatter(tmp_v, [jnp.full((16,), e, jnp.int32)],
                                   csum, mask=last_lane)
            acc = tmp_v[...] + fc_v.at[s][pl.ds(gb, 16)]
            ob_v.at[s][pl.ds(gb, 16)] = 1.0 / (1.0 + jnp.exp(-acc))
            return gcarry

        lax.fori_loop(0, GROUPS, group_body, 0)

    issue(0, 0)
    issue(1, 1)

    def j_body(j, carry):
        for s in range(NBUF):
            i = j * NBUF + s

            @pl.when(j >= 1)
            def _():
                out_copy(i - NBUF, s).wait()

            wait_in(i, s)
            s2 = (s + 2) % NBUF
            if s < NBUF - 2:
                issue(i + 2, s2)
            else:
                @pl.when(j <= NUM_CHUNKS // NBUF - 2)
                def _():
                    issue(i + 2, s2)
            compute(i, s)
            out_copy(i, s).start()
        return carry

    lax.fori_loop(0, NUM_CHUNKS // NBUF, j_body, 0)
    for s in range(NBUF):
        out_copy(NUM_CHUNKS - NBUF + s, s).wait()


def kernel(embedding, src_id, dst_id, edge_feats, W, b):
    w128 = W[:D_EMB, 0].reshape(1, D_EMB)
    es = _scale_table(embedding, w128)
    w6 = W[D_EMB:, 0]
    sw = jnp.kron(jnp.eye(16, dtype=jnp.float32), w6.reshape(D_FEAT, 1))
    b16 = jnp.broadcast_to(b, (1, 16))
    fc = _featdot(edge_feats.reshape(N_EDGES // 16, 16 * D_FEAT),
                  sw, b16).reshape(-1)
    out = _edge_kernel(
        es, embedding,
        src_id.astype(jnp.int32), dst_id.astype(jnp.int32), fc)
    return out.reshape(N_EDGES, 1)
